# initial kernel scaffold (unmeasured)
import jax
import jax.numpy as jnp
from jax import lax
from jax.experimental import pallas as pl
from jax.experimental.pallas import tpu as pltpu


def kernel(
    x,
):
    def body(*refs):
        pass

    out_shape = jax.ShapeDtypeStruct(..., jnp.float32)
    return pl.pallas_call(body, out_shape=out_shape)(...)


# baseline (device time: 33527 ns/iter reference)
import jax
import jax.numpy as jnp
from jax import lax
from jax.experimental import pallas as pl
from jax.experimental.pallas import tpu as pltpu

M = 2048
N = 1024
HALF_M = M // 2
HALF_N = N // 2


def kernel(x):
    def body(x_ref, out_ref, ysend, yrecv, sems):
        my_x = lax.axis_index("x")
        my_y = lax.axis_index("y")

        barrier = pltpu.get_barrier_semaphore()
        pl.semaphore_signal(
            barrier, inc=1, device_id=(my_x, 1 - my_y),
            device_id_type=pl.DeviceIdType.MESH,
        )
        pl.semaphore_signal(
            barrier, inc=1, device_id=(1 - my_x, my_y),
            device_id_type=pl.DeviceIdType.MESH,
        )
        pl.semaphore_wait(barrier, 2)

        row0 = my_x * HALF_M
        col_me = my_y * HALF_N
        col_peer = (1 - my_y) * HALF_N

        ysend[...] = x_ref[
            0, pl.ds(row0, HALF_M), pl.ds(col_peer, HALF_N)
        ].astype(jnp.bfloat16)
        rdma_y = pltpu.make_async_remote_copy(
            src_ref=ysend,
            dst_ref=yrecv,
            send_sem=sems.at[0],
            recv_sem=sems.at[1],
            device_id=(my_x, 1 - my_y),
            device_id_type=pl.DeviceIdType.MESH,
        )
        rdma_y.start()
        rdma_y.wait()

        acc = (
            x_ref[0, pl.ds(row0, HALF_M), pl.ds(col_me, HALF_N)]
            + yrecv[...].astype(jnp.float32)
        )
        out_ref[pl.ds(row0, HALF_M), :] = acc.astype(jnp.bfloat16)

        rdma_x = pltpu.make_async_remote_copy(
            src_ref=out_ref.at[pl.ds(row0, HALF_M), :],
            dst_ref=out_ref.at[pl.ds(row0, HALF_M), :],
            send_sem=sems.at[2],
            recv_sem=sems.at[3],
            device_id=(1 - my_x, my_y),
            device_id_type=pl.DeviceIdType.MESH,
        )
        rdma_x.start()
        rdma_x.wait()

    return pl.pallas_call(
        body,
        out_shape=jax.ShapeDtypeStruct((M, HALF_N), jnp.bfloat16),
        in_specs=[pl.BlockSpec(memory_space=pltpu.VMEM)],
        out_specs=pl.BlockSpec(memory_space=pltpu.VMEM),
        scratch_shapes=[
            pltpu.VMEM((HALF_M, HALF_N), jnp.bfloat16),
            pltpu.VMEM((HALF_M, HALF_N), jnp.bfloat16),
            pltpu.SemaphoreType.DMA((4,)),
        ],
        compiler_params=pltpu.CompilerParams(collective_id=0),
    )(x)


# device time: 23728 ns/iter; 1.4130x vs baseline; 1.4130x over previous
import jax
import jax.numpy as jnp
from jax import lax
from jax.experimental import pallas as pl
from jax.experimental.pallas import tpu as pltpu

M = 2048
N = 1024
HALF_M = M // 2
HALF_N = N // 2
C = 8
CHUNK = HALF_M // C


def kernel(x):
    def body(x_ref, out_ref, ysend, yrecv, ysend_sems, yrecv_sems,
             xsend_sems, xrecv_sems):
        my_x = lax.axis_index("x")
        my_y = lax.axis_index("y")

        barrier = pltpu.get_barrier_semaphore()
        pl.semaphore_signal(
            barrier, inc=1, device_id=(my_x, 1 - my_y),
            device_id_type=pl.DeviceIdType.MESH,
        )
        pl.semaphore_signal(
            barrier, inc=1, device_id=(1 - my_x, my_y),
            device_id_type=pl.DeviceIdType.MESH,
        )
        pl.semaphore_wait(barrier, 2)

        row0 = my_x * HALF_M
        col_me = my_y * HALF_N
        col_peer = (1 - my_y) * HALF_N

        y_rdmas = []
        for k in range(C):
            ysend[pl.ds(k * CHUNK, CHUNK), :] = x_ref[
                0, pl.ds(row0 + k * CHUNK, CHUNK), pl.ds(col_peer, HALF_N)
            ].astype(jnp.bfloat16)
            rdma = pltpu.make_async_remote_copy(
                src_ref=ysend.at[pl.ds(k * CHUNK, CHUNK), :],
                dst_ref=yrecv.at[pl.ds(k * CHUNK, CHUNK), :],
                send_sem=ysend_sems.at[k],
                recv_sem=yrecv_sems.at[k],
                device_id=(my_x, 1 - my_y),
                device_id_type=pl.DeviceIdType.MESH,
            )
            rdma.start()
            y_rdmas.append(rdma)

        x_rdmas = []
        for k in range(C):
            y_rdmas[k].wait_recv()
            acc = (
                x_ref[0, pl.ds(row0 + k * CHUNK, CHUNK), pl.ds(col_me, HALF_N)]
                + yrecv[pl.ds(k * CHUNK, CHUNK), :].astype(jnp.float32)
            )
            out_ref[pl.ds(row0 + k * CHUNK, CHUNK), :] = acc.astype(jnp.bfloat16)
            rdma = pltpu.make_async_remote_copy(
                src_ref=out_ref.at[pl.ds(row0 + k * CHUNK, CHUNK), :],
                dst_ref=out_ref.at[pl.ds(row0 + k * CHUNK, CHUNK), :],
                send_sem=xsend_sems.at[k],
                recv_sem=xrecv_sems.at[k],
                device_id=(1 - my_x, my_y),
                device_id_type=pl.DeviceIdType.MESH,
            )
            rdma.start()
            x_rdmas.append(rdma)

        for k in range(C):
            y_rdmas[k].wait_send()
            x_rdmas[k].wait()

    return pl.pallas_call(
        body,
        out_shape=jax.ShapeDtypeStruct((M, HALF_N), jnp.bfloat16),
        in_specs=[pl.BlockSpec(memory_space=pltpu.VMEM)],
        out_specs=pl.BlockSpec(memory_space=pltpu.VMEM),
        scratch_shapes=[
            pltpu.VMEM((HALF_M, HALF_N), jnp.bfloat16),
            pltpu.VMEM((HALF_M, HALF_N), jnp.bfloat16),
            pltpu.SemaphoreType.DMA((C,)),
            pltpu.SemaphoreType.DMA((C,)),
            pltpu.SemaphoreType.DMA((C,)),
            pltpu.SemaphoreType.DMA((C,)),
        ],
        compiler_params=pltpu.CompilerParams(collective_id=0),
    )(x)
